# Initial kernel scaffold; baseline (speedup 1.0000x reference)
import jax
import jax.numpy as jnp
from jax.experimental import pallas as pl


def kernel(t, sqrt_alpha_bar, sqrt_one_minus_alpha_bar):
    raise NotImplementedError("write your pallas kernel here")


# speedup vs baseline: 8.2925x; 8.2925x over previous
import jax
import jax.numpy as jnp
from jax import lax
from jax.experimental import pallas as pl
from jax.experimental.pallas import tpu as pltpu
from jax.experimental.pallas import tpu_sc as plsc

_BATCH = 16384
_TAB = 1024
_NC, _NS = 2, 16
_NW = _NC * _NS
_BPW = _BATCH // _NW
_L = 16


def _body(t_hbm, a_hbm, s_hbm, out_a, out_s, ta_v, ts_v, idx_v, oa_v, os_v):
    wid = lax.axis_index("s") * _NC + lax.axis_index("c")
    base = wid * _BPW
    pltpu.sync_copy(a_hbm, ta_v)
    pltpu.sync_copy(s_hbm, ts_v)
    pltpu.sync_copy(t_hbm.at[pl.ds(base, _BPW)], idx_v)
    for j in range(_BPW // _L):
        iv = idx_v[pl.ds(j * _L, _L)]
        oa_v[pl.ds(j * _L, _L)] = plsc.load_gather(ta_v, [iv])
        os_v[pl.ds(j * _L, _L)] = plsc.load_gather(ts_v, [iv])
    pltpu.sync_copy(oa_v, out_a.at[pl.ds(base, _BPW)])
    pltpu.sync_copy(os_v, out_s.at[pl.ds(base, _BPW)])


def kernel(t, sqrt_alpha_bar, sqrt_one_minus_alpha_bar):
    t32 = t.astype(jnp.int32)
    pad = _TAB - sqrt_alpha_bar.shape[0]
    a = jnp.pad(sqrt_alpha_bar.astype(jnp.float32), (0, pad))
    s = jnp.pad(sqrt_one_minus_alpha_bar.astype(jnp.float32), (0, pad))
    run = pl.kernel(
        _body,
        out_type=(
            jax.ShapeDtypeStruct((_BATCH,), jnp.float32),
            jax.ShapeDtypeStruct((_BATCH,), jnp.float32),
        ),
        mesh=plsc.VectorSubcoreMesh(core_axis_name="c", subcore_axis_name="s"),
        compiler_params=pltpu.CompilerParams(needs_layout_passes=False),
        scratch_types=[
            pltpu.VMEM((_TAB,), jnp.float32),
            pltpu.VMEM((_TAB,), jnp.float32),
            pltpu.VMEM((_BPW,), jnp.int32),
            pltpu.VMEM((_BPW,), jnp.float32),
            pltpu.VMEM((_BPW,), jnp.float32),
        ],
    )
    return run(t32, a, s)


# speedup vs baseline: 8.3167x; 1.0029x over previous
import jax
import jax.numpy as jnp
from jax import lax
from jax.experimental import pallas as pl
from jax.experimental.pallas import tpu as pltpu
from jax.experimental.pallas import tpu_sc as plsc

_BATCH = 16384
_TAB = 1000
_NC, _NS = 2, 16
_NW = _NC * _NS
_BPW = _BATCH // _NW
_L = 16


def _body(t_hbm, a_hbm, s_hbm, out_a, out_s, ta_v, ts_v, idx_v, oa_v, os_v):
    wid = lax.axis_index("s") * _NC + lax.axis_index("c")
    base = wid * _BPW
    pltpu.sync_copy(a_hbm, ta_v)
    pltpu.sync_copy(s_hbm, ts_v)
    pltpu.sync_copy(t_hbm.at[pl.ds(base, _BPW)], idx_v)
    for j in range(_BPW // _L):
        iv = idx_v[pl.ds(j * _L, _L)]
        oa_v[pl.ds(j * _L, _L)] = plsc.load_gather(ta_v, [iv])
        os_v[pl.ds(j * _L, _L)] = plsc.load_gather(ts_v, [iv])
    pltpu.sync_copy(oa_v, out_a.at[pl.ds(base, _BPW)])
    pltpu.sync_copy(os_v, out_s.at[pl.ds(base, _BPW)])


def kernel(t, sqrt_alpha_bar, sqrt_one_minus_alpha_bar):
    t32 = t.astype(jnp.int32)
    a = sqrt_alpha_bar.astype(jnp.float32)
    s = sqrt_one_minus_alpha_bar.astype(jnp.float32)
    run = pl.kernel(
        _body,
        out_type=(
            jax.ShapeDtypeStruct((_BATCH,), jnp.float32),
            jax.ShapeDtypeStruct((_BATCH,), jnp.float32),
        ),
        mesh=plsc.VectorSubcoreMesh(core_axis_name="c", subcore_axis_name="s"),
        compiler_params=pltpu.CompilerParams(needs_layout_passes=False),
        scratch_types=[
            pltpu.VMEM((_TAB,), jnp.float32),
            pltpu.VMEM((_TAB,), jnp.float32),
            pltpu.VMEM((_BPW,), jnp.int32),
            pltpu.VMEM((_BPW,), jnp.float32),
            pltpu.VMEM((_BPW,), jnp.float32),
        ],
    )
    return run(t32, a, s)


# speedup vs baseline: 8.7913x; 1.0571x over previous
import jax
import jax.numpy as jnp
from jax import lax
from jax.experimental import pallas as pl
from jax.experimental.pallas import tpu as pltpu
from jax.experimental.pallas import tpu_sc as plsc

_BATCH = 16384
_TAB = 1000
_NC, _NS = 2, 16
_NW = _NC * _NS
_BPW = _BATCH // _NW
_L = 16


def _body(t_hbm, a_hbm, s_hbm, out_a, out_s,
          ta_v, ts_v, idx_v, oa_v, os_v, sem_in, sem_out):
    wid = lax.axis_index("s") * _NC + lax.axis_index("c")
    base = wid * _BPW
    ca = pltpu.make_async_copy(a_hbm, ta_v, sem_in)
    cs = pltpu.make_async_copy(s_hbm, ts_v, sem_in)
    ci = pltpu.make_async_copy(t_hbm.at[pl.ds(base, _BPW)], idx_v, sem_in)
    ca.start()
    cs.start()
    ci.start()
    ca.wait()
    cs.wait()
    ci.wait()
    for j in range(_BPW // _L):
        iv = idx_v[pl.ds(j * _L, _L)]
        oa_v[pl.ds(j * _L, _L)] = plsc.load_gather(ta_v, [iv])
        os_v[pl.ds(j * _L, _L)] = plsc.load_gather(ts_v, [iv])
    coa = pltpu.make_async_copy(oa_v, out_a.at[pl.ds(base, _BPW)], sem_out)
    cos = pltpu.make_async_copy(os_v, out_s.at[pl.ds(base, _BPW)], sem_out)
    coa.start()
    cos.start()
    coa.wait()
    cos.wait()


def kernel(t, sqrt_alpha_bar, sqrt_one_minus_alpha_bar):
    t32 = t.astype(jnp.int32)
    a = sqrt_alpha_bar.astype(jnp.float32)
    s = sqrt_one_minus_alpha_bar.astype(jnp.float32)
    run = pl.kernel(
        _body,
        out_type=(
            jax.ShapeDtypeStruct((_BATCH,), jnp.float32),
            jax.ShapeDtypeStruct((_BATCH,), jnp.float32),
        ),
        mesh=plsc.VectorSubcoreMesh(core_axis_name="c", subcore_axis_name="s"),
        compiler_params=pltpu.CompilerParams(needs_layout_passes=False),
        scratch_types=[
            pltpu.VMEM((_TAB,), jnp.float32),
            pltpu.VMEM((_TAB,), jnp.float32),
            pltpu.VMEM((_BPW,), jnp.int32),
            pltpu.VMEM((_BPW,), jnp.float32),
            pltpu.VMEM((_BPW,), jnp.float32),
            pltpu.SemaphoreType.DMA,
            pltpu.SemaphoreType.DMA,
        ],
    )
    return run(t32, a, s)
